# Initial kernel scaffold; baseline (speedup 1.0000x reference)
import jax
import jax.numpy as jnp
from jax.experimental import pallas as pl


def kernel(user, item, time, user_table, item_table, time_table, Wu, bu, Wi, bi, Wt, bt):
    raise NotImplementedError("write your pallas kernel here")


# speedup vs baseline: 4.3344x; 4.3344x over previous
import functools

import jax
import jax.numpy as jnp
from jax import lax
from jax.experimental import pallas as pl
from jax.experimental.pallas import tpu as pltpu
from jax.experimental.pallas import tpu_sc as plsc

B = 16384
D = 128
NC, NS = 2, 16
NW = NC * NS
CHUNK = 128
CPW = B // (NW * CHUNK)


def _gather3(u_idx, i_idx, t_idx, user_table, item_table, time_table):
    mesh = plsc.VectorSubcoreMesh(core_axis_name="c", subcore_axis_name="s")
    out_type = [jax.ShapeDtypeStruct((B, D), jnp.float32)] * 3

    @functools.partial(
        pl.kernel,
        mesh=mesh,
        out_type=out_type,
        scratch_types=[
            pltpu.VMEM((CHUNK,), jnp.int32),
            pltpu.VMEM((CHUNK, D), jnp.float32),
            pltpu.SemaphoreType.DMA,
        ],
    )
    def gather_kernel(u_idx, i_idx, t_idx, utab, itab, ttab,
                      uo, io, to, idx_v, rows_v, sem):
        wid = lax.axis_index("s") * NC + lax.axis_index("c")
        for idxs, tab, out in ((u_idx, utab, uo), (i_idx, itab, io),
                               (t_idx, ttab, to)):
            for j in range(CPW):
                row = wid * CPW + j
                pltpu.sync_copy(idxs.at[row], idx_v)
                pltpu.async_copy(tab.at[idx_v], rows_v, sem).wait()
                pltpu.sync_copy(rows_v, out.at[pl.ds(row * CHUNK, CHUNK)])

    return gather_kernel(u_idx, i_idx, t_idx, user_table, item_table,
                         time_table)


BB = 1024


def _dense_body(u_ref, i_ref, t_ref, Wu_ref, bu_ref, Wi_ref, bi_ref,
                Wt_ref, bt_ref, o_ref):
    dn = (((1,), (1,)), ((), ()))
    a = lax.dot_general(u_ref[...], Wu_ref[...], dn,
                        preferred_element_type=jnp.float32) + bu_ref[...]
    b = lax.dot_general(i_ref[...], Wi_ref[...], dn,
                        preferred_element_type=jnp.float32) + bi_ref[...]
    c = lax.dot_general(t_ref[...], Wt_ref[...], dn,
                        preferred_element_type=jnp.float32) + bt_ref[...]
    o_ref[...] = jax.nn.sigmoid(jnp.sum(a * b * c, axis=-1))


def _dense(u_rows, i_rows, t_rows, Wu, bu, Wi, bi, Wt, bt):
    grid = (B // BB,)
    row_spec = pl.BlockSpec((BB, D), lambda i: (i, 0))
    w_spec = pl.BlockSpec((D, D), lambda i: (0, 0))
    b_spec = pl.BlockSpec((D,), lambda i: (0,))
    return pl.pallas_call(
        _dense_body,
        grid=grid,
        in_specs=[row_spec, row_spec, row_spec,
                  w_spec, b_spec, w_spec, b_spec, w_spec, b_spec],
        out_specs=pl.BlockSpec((BB,), lambda i: (i,)),
        out_shape=jax.ShapeDtypeStruct((B,), jnp.float32),
    )(u_rows, i_rows, t_rows, Wu, bu, Wi, bi, Wt, bt)


def kernel(user, item, time, user_table, item_table, time_table,
           Wu, bu, Wi, bi, Wt, bt):
    u_idx = user.astype(jnp.int32).reshape(B // CHUNK, CHUNK)
    i_idx = item.astype(jnp.int32).reshape(B // CHUNK, CHUNK)
    t_idx = time.astype(jnp.int32).reshape(B // CHUNK, CHUNK)
    u_rows, i_rows, t_rows = _gather3(u_idx, i_idx, t_idx,
                                      user_table, item_table, time_table)
    return _dense(u_rows, i_rows, t_rows, Wu, bu, Wi, bi, Wt, bt)


# speedup vs baseline: 5.2092x; 1.2018x over previous
import functools

import jax
import jax.numpy as jnp
from jax import lax
from jax.experimental import pallas as pl
from jax.experimental.pallas import tpu as pltpu
from jax.experimental.pallas import tpu_sc as plsc

B = 16384
D = 128
NC, NS = 2, 16
NW = NC * NS
CHUNK = 128
CPW = B // (NW * CHUNK)


def _gather3(u_idx, i_idx, t_idx, user_table, item_table, time_table):
    mesh = plsc.VectorSubcoreMesh(core_axis_name="c", subcore_axis_name="s")
    out_type = [jax.ShapeDtypeStruct((B, D), jnp.float32)] * 3
    NBUF = 4
    NCH = 3 * CPW

    @functools.partial(
        pl.kernel,
        mesh=mesh,
        out_type=out_type,
        scratch_types=[
            pltpu.VMEM((3, CPW, CHUNK), jnp.int32),
            pltpu.VMEM((NBUF, CHUNK, D), jnp.float32),
            pltpu.SemaphoreType.DMA,
            pltpu.SemaphoreType.DMA,
            pltpu.SemaphoreType.DMA,
        ],
    )
    def gather_kernel(u_idx, i_idx, t_idx, utab, itab, ttab,
                      uo, io, to, idx3, rows, sem_i, sem_g, sem_w):
        wid = lax.axis_index("s") * NC + lax.axis_index("c")
        idxs = (u_idx, i_idx, t_idx)
        tabs = (utab, itab, ttab)
        outs = (uo, io, to)
        cps = [pltpu.async_copy(idxs[t].at[pl.ds(wid * CPW, CPW)],
                                idx3.at[t], sem_i) for t in range(3)]
        for cp in cps:
            cp.wait()

        def tj(c):
            return c % 3, c // 3

        gat = [None] * NCH
        wrt = [None] * NCH

        def start_gather(c):
            t, j = tj(c)
            gat[c] = pltpu.async_copy(tabs[t].at[idx3.at[t, j]],
                                      rows.at[c % NBUF], sem_g)

        start_gather(0)
        for c in range(NCH):
            if c + 1 < NCH:
                if c + 1 >= NBUF:
                    wrt[c + 1 - NBUF].wait()
                start_gather(c + 1)
            gat[c].wait()
            t, j = tj(c)
            row = wid * CPW + j
            wrt[c] = pltpu.async_copy(rows.at[c % NBUF],
                                      outs[t].at[pl.ds(row * CHUNK, CHUNK)],
                                      sem_w)
        for c in range(NCH - NBUF, NCH):
            wrt[c].wait()

    return gather_kernel(u_idx, i_idx, t_idx, user_table, item_table,
                         time_table)


BB = 1024


def _dense_body(u_ref, i_ref, t_ref, Wu_ref, bu_ref, Wi_ref, bi_ref,
                Wt_ref, bt_ref, o_ref):
    dn = (((1,), (1,)), ((), ()))
    a = lax.dot_general(u_ref[...], Wu_ref[...], dn,
                        preferred_element_type=jnp.float32) + bu_ref[...]
    b = lax.dot_general(i_ref[...], Wi_ref[...], dn,
                        preferred_element_type=jnp.float32) + bi_ref[...]
    c = lax.dot_general(t_ref[...], Wt_ref[...], dn,
                        preferred_element_type=jnp.float32) + bt_ref[...]
    o_ref[...] = jax.nn.sigmoid(jnp.sum(a * b * c, axis=-1))


def _dense(u_rows, i_rows, t_rows, Wu, bu, Wi, bi, Wt, bt):
    grid = (B // BB,)
    row_spec = pl.BlockSpec((BB, D), lambda i: (i, 0))
    w_spec = pl.BlockSpec((D, D), lambda i: (0, 0))
    b_spec = pl.BlockSpec((D,), lambda i: (0,))
    return pl.pallas_call(
        _dense_body,
        grid=grid,
        in_specs=[row_spec, row_spec, row_spec,
                  w_spec, b_spec, w_spec, b_spec, w_spec, b_spec],
        out_specs=pl.BlockSpec((BB,), lambda i: (i,)),
        out_shape=jax.ShapeDtypeStruct((B,), jnp.float32),
    )(u_rows, i_rows, t_rows, Wu, bu, Wi, bi, Wt, bt)


def kernel(user, item, time, user_table, item_table, time_table,
           Wu, bu, Wi, bi, Wt, bt):
    u_idx = user.astype(jnp.int32).reshape(B // CHUNK, CHUNK)
    i_idx = item.astype(jnp.int32).reshape(B // CHUNK, CHUNK)
    t_idx = time.astype(jnp.int32).reshape(B // CHUNK, CHUNK)
    u_rows, i_rows, t_rows = _gather3(u_idx, i_idx, t_idx,
                                      user_table, item_table, time_table)
    return _dense(u_rows, i_rows, t_rows, Wu, bu, Wi, bi, Wt, bt)
